# baseline (speedup 1.0000x reference)
import functools

import jax
import jax.numpy as jnp
from jax.experimental import pallas as pl
from jax.experimental.pallas import tpu as pltpu, tpu_sc as plsc

VQN = 512
VQD = 64
NGROUPS = 16
NCARD = 8
EMA_ALPHA = 0.99
EPS = 1e-5


def _conv_rows(x, w_taps, k):
    T = x.shape[0]
    pad = k // 2
    xp = jnp.pad(x.astype(jnp.bfloat16), ((pad, pad), (0, 0)))
    acc = jax.lax.dot_general(
        xp[0:T, :], w_taps[0].astype(jnp.bfloat16),
        (((1,), (0,)), ((), ())), preferred_element_type=jnp.float32)
    for dt in range(1, k):
        acc = acc + jax.lax.dot_general(
            xp[dt:dt + T, :], w_taps[dt].astype(jnp.bfloat16),
            (((1,), (0,)), ((), ())), preferred_element_type=jnp.float32)
    return acc


def _conv_rows_im2col(x, w_taps, k):
    T, Cin = x.shape
    pad = k // 2
    xp = jnp.pad(x.astype(jnp.bfloat16), ((pad, pad), (0, 0)))
    xs = jnp.concatenate([xp[dt:dt + T, :] for dt in range(k)], axis=1)
    wf = w_taps[...].reshape(k * Cin, w_taps.shape[2]).astype(jnp.bfloat16)
    return jax.lax.dot_general(xs, wf, (((1,), (0,)), ((), ())),
                               preferred_element_type=jnp.float32)


def _gn_rows(h, g, b, egrp):
    T, C = h.shape
    n = (C // NGROUPS) * T
    hp = jax.lax.Precision.HIGHEST
    s_c = jnp.sum(h, axis=0, keepdims=True)
    s_g = jax.lax.dot_general(s_c, egrp, (((1,), (0,)), ((), ())),
                              precision=hp,
                              preferred_element_type=jnp.float32)
    m_c = jax.lax.dot_general(s_g, egrp, (((1,), (1,)), ((), ())),
                              precision=hp,
                              preferred_element_type=jnp.float32) / n
    d = h - m_c
    v_c = jnp.sum(d * d, axis=0, keepdims=True)
    v_g = jax.lax.dot_general(v_c, egrp, (((1,), (0,)), ((), ())),
                              precision=hp,
                              preferred_element_type=jnp.float32)
    v_cb = jax.lax.dot_general(v_g, egrp, (((1,), (1,)), ((), ())),
                               precision=hp,
                               preferred_element_type=jnp.float32) / n
    return d / jnp.sqrt(v_cb + EPS) * g[None, :] + b[None, :]


def _maxpool2_rows(h):
    T, C = h.shape
    return jnp.max(h.reshape(T // 2, 2, C), axis=1)


def _upsample2_rows(h):
    T, C = h.shape
    return jnp.broadcast_to(h[:, None, :], (T, 2, C)).reshape(2 * T, C)


def _encoder_body(x_ref, w0t_ref, g0_ref, b0_ref, wxd_ref, gx_ref, bx_ref,
                  w2t_ref, egrp_ref, out_ref):
    x = x_ref[0]
    xt = x.T
    egrp = egrp_ref[...]
    g0 = g0_ref[...]
    b0 = b0_ref[...]
    gx = gx_ref[...]
    bx = bx_ref[...]

    h = _conv_rows_im2col(xt, w0t_ref, 5)
    h = jax.nn.relu(_gn_rows(h, g0, b0, egrp))
    h = jax.nn.relu(h + _gn_rows(_conv_rows_im2col(h, wxd_ref, 3),
                                 gx, bx, egrp))
    h = _maxpool2_rows(h)
    h = jax.nn.relu(h + _gn_rows(_conv_rows_im2col(h, wxd_ref, 3),
                                 gx, bx, egrp))
    h = _maxpool2_rows(h)
    out_ref[0] = _conv_rows_im2col(h, w2t_ref, 3)


def _argmin_body(h2_ref, g2_ref, b2_ref, w_ref, zaug_ref, idx_ref):
    B, Tq, C = h2_ref.shape
    N = B * Tq
    Z0 = h2_ref[...].reshape(N, C)
    m = jnp.mean(Z0, axis=0, keepdims=True)
    d = Z0 - m
    v = jnp.mean(d * d, axis=0, keepdims=True)
    Z = d / jnp.sqrt(v + EPS) * g2_ref[...][None, :] + b2_ref[...][None, :]

    w = w_ref[...]
    hp = jax.lax.Precision.HIGHEST
    zz = jnp.sum(Z * Z, axis=1, keepdims=True)
    zw = jax.lax.dot_general(Z.astype(jnp.bfloat16), w.astype(jnp.bfloat16),
                             (((1,), (1,)), ((), ())),
                             preferred_element_type=jnp.float32)
    ww = jax.lax.dot_general(jnp.ones((1, C), jnp.float32), w * w,
                             (((1,), (1,)), ((), ())), precision=hp,
                             preferred_element_type=jnp.float32)
    dist = zz - 2.0 * zw + ww
    amin = jnp.min(dist, axis=1, keepdims=True)
    lane = jax.lax.broadcasted_iota(jnp.int32, (N, VQN), 1)
    idx = jnp.min(jnp.where(dist == amin, lane, jnp.int32(2 ** 30)),
                  axis=1, keepdims=True)
    idx_ref[...] = idx

    alane = jax.lax.broadcasted_iota(jnp.int32, (N, AW), 1)
    zaug_ref[...] = (jnp.pad(Z, ((0, 0), (0, AW - C)))
                     + (alane == C).astype(jnp.float32))


def _ema_body(parts_ref, w_ref, c_ref, wpad_ref):
    acc = parts_ref[0] + parts_ref[1]
    counts = acc[:, VQD:VQD + 1]
    sums = acc[:, :VQD]
    w = w_ref[...]
    c = c_ref[...]
    mask = counts > 0
    c_new = jnp.where(mask, EMA_ALPHA * c + (1.0 - EMA_ALPHA) * counts, c)
    w_new = jnp.where(
        mask,
        EMA_ALPHA * w + (1.0 - EMA_ALPHA) * sums / jnp.maximum(c_new, 1e-12),
        w)
    wpad_ref[...] = jnp.pad(w_new, ((0, 0), (0, AW - VQD)))


NTOK = 4096
AW = 128
SC_NC = 2
SC_NS = 16
SC_NW = SC_NC * SC_NS
SC_RPW = NTOK // SC_NW
SC_CPS = VQN // SC_NS

_sc_mesh = plsc.VectorSubcoreMesh(core_axis_name="c", subcore_axis_name="s")


@functools.partial(
    pl.kernel, mesh=_sc_mesh,
    out_type=jax.ShapeDtypeStruct((SC_NC, VQN, AW), jnp.float32),
    scratch_types=[
        pltpu.VMEM((SC_RPW,), jnp.int32),
        pltpu.VMEM((SC_RPW, AW), jnp.float32),
        pltpu.VMEM((SC_CPS, AW), jnp.float32),
        pltpu.VMEM_SHARED((VQN, AW), jnp.float32),
        pltpu.SemaphoreType.DMA,
    ],
)
def _sc_scatter_add(idx_hbm, zaug_hbm, zero_hbm, out_hbm,
                    idx_v, zaug_v, acc_v, acc_sh, sem):
    cid = jax.lax.axis_index("c")
    sid = jax.lax.axis_index("s")
    wid = sid * SC_NC + cid

    pltpu.sync_copy(zero_hbm.at[pl.ds(sid * SC_CPS, SC_CPS)],
                    acc_sh.at[pl.ds(sid * SC_CPS, SC_CPS)])
    plsc.subcore_barrier()

    base = wid * SC_RPW
    pltpu.sync_copy(idx_hbm.at[pl.ds(base, SC_RPW)], idx_v)
    pltpu.sync_copy(zaug_hbm.at[pl.ds(base, SC_RPW)], zaug_v)
    pltpu.sync_copy(zaug_v, acc_sh.at[idx_v], add=True)
    plsc.subcore_barrier()

    pltpu.sync_copy(acc_sh.at[pl.ds(sid * SC_CPS, SC_CPS)], acc_v)
    pltpu.sync_copy(acc_v, out_hbm.at[cid].at[pl.ds(sid * SC_CPS, SC_CPS)])


@functools.partial(
    pl.kernel, mesh=_sc_mesh,
    out_type=jax.ShapeDtypeStruct((NTOK, AW), jnp.float32),
    scratch_types=[
        pltpu.VMEM((SC_RPW,), jnp.int32),
        pltpu.VMEM((SC_RPW, AW), jnp.float32),
        pltpu.SemaphoreType.DMA,
    ],
)
def _sc_gather(table_hbm, idx_hbm, out_hbm, idx_v, rows_v, sem):
    cid = jax.lax.axis_index("c")
    sid = jax.lax.axis_index("s")
    wid = sid * SC_NC + cid
    base = wid * SC_RPW
    pltpu.sync_copy(idx_hbm.at[pl.ds(base, SC_RPW)], idx_v)
    pltpu.async_copy(table_hbm.at[idx_v], rows_v, sem).wait()
    pltpu.sync_copy(rows_v, out_hbm.at[pl.ds(base, SC_RPW)])


def _decoder_body(zaug_ref, wj_ref, wd2t_ref, gd2_ref, bd2_ref, wd1at_ref,
                  gd1a_ref, bd1a_ref, wd1bt_ref, gd1b_ref, bd1b_ref,
                  wd0t_ref, bd0_ref, egrp_ref, out_ref, commit_ref):
    egrp = egrp_ref[...]
    gd2 = gd2_ref[...]
    bd2 = bd2_ref[...]
    gd1a = gd1a_ref[...]
    bd1a = bd1a_ref[...]
    gd1b = gd1b_ref[...]
    bd1b = bd1b_ref[...]

    def deres1(t):
        o = jax.nn.relu(_gn_rows(_conv_rows(t, wd1at_ref, 3), gd1a, bd1a, egrp))
        o = _gn_rows(_conv_rows(o, wd1bt_ref, 3), gd1b, bd1b, egrp)
        return jax.nn.relu(t + o)

    b = pl.program_id(0)
    nb = pl.num_programs(0)
    z = zaug_ref[0][:, :VQD]
    hq = wj_ref[0][:, :VQD]
    dz = z - hq
    part = jnp.sum(dz * dz)

    @pl.when(b == 0)
    def _():
        commit_ref[...] = jnp.zeros((1, 1), jnp.float32)

    commit_ref[...] += part.reshape(1, 1)

    @pl.when(b == nb - 1)
    def _():
        commit_ref[...] = commit_ref[...] / (nb * zaug_ref.shape[1])

    t = jax.nn.relu(_gn_rows(_conv_rows(hq, wd2t_ref, 3),
                             gd2, bd2, egrp))
    t = _upsample2_rows(t)
    t = deres1(t)
    t = _upsample2_rows(t)
    t = deres1(t)
    r = _conv_rows(t, wd0t_ref, 5) + bd0_ref[...][None, :]
    out_ref[0] = r.T


def _full(bspec):
    return pl.BlockSpec(bspec, lambda *_: (0,) * len(bspec))


def _block_diag_taps(wx):
    co, ci_g, k = wx.shape
    gsz = ci_g
    ngr = co // gsz
    wt = jnp.transpose(wx, (2, 0, 1))
    out = jnp.zeros((k, co, co), jnp.float32)
    for g in range(ngr):
        blk = jnp.transpose(wt[:, g * gsz:(g + 1) * gsz, :], (0, 2, 1))
        out = jax.lax.dynamic_update_slice(out, blk, (0, g * gsz, g * gsz))
    return out


def kernel(x, W0, g0, b0, Wx, gx, bx, W2, g2, b2, Wd2, gd2, bd2, Wd1a, gd1a,
           bd1a, Wd1b, gd1b, bd1b, Wd0, bd0, dict_val, dict_cnt):
    B, Cx, T = x.shape
    Tq = T // 4

    w0t = jnp.transpose(W0, (2, 1, 0))
    wxd = _block_diag_taps(Wx)
    w2t = jnp.transpose(W2, (2, 1, 0))
    wd2t = jnp.transpose(Wd2, (2, 0, 1))
    wd1at = jnp.transpose(Wd1a, (2, 0, 1))
    wd1bt = jnp.transpose(Wd1b, (2, 0, 1))
    wd0t = jnp.transpose(Wd0, (2, 0, 1))
    ch = jax.lax.broadcasted_iota(jnp.int32, (256, NGROUPS), 0)
    gr = jax.lax.broadcasted_iota(jnp.int32, (256, NGROUPS), 1)
    egrp = (ch // (256 // NGROUPS) == gr).astype(jnp.float32)

    h2 = pl.pallas_call(
        _encoder_body,
        grid=(B,),
        in_specs=[
            pl.BlockSpec((1, Cx, T), lambda b: (b, 0, 0)),
            _full((5, Cx, 256)), _full((256,)), _full((256,)),
            _full((3, 256, 256)), _full((256,)), _full((256,)),
            _full((3, 256, VQD)), _full((256, NGROUPS)),
        ],
        out_specs=pl.BlockSpec((1, Tq, VQD), lambda b: (b, 0, 0)),
        out_shape=jax.ShapeDtypeStruct((B, Tq, VQD), jnp.float32),
    )(x, w0t, g0, b0, wxd, gx, bx, w2t, egrp)

    zaug, idxm = pl.pallas_call(
        _argmin_body,
        in_specs=[
            _full((B, Tq, VQD)), _full((VQD,)), _full((VQD,)),
            _full((VQN, VQD)),
        ],
        out_specs=[_full((NTOK, AW)), _full((NTOK, 1))],
        out_shape=[
            jax.ShapeDtypeStruct((NTOK, AW), jnp.float32),
            jax.ShapeDtypeStruct((NTOK, 1), jnp.int32),
        ],
    )(h2, g2, b2, dict_val)

    idxf = idxm.reshape(NTOK)
    parts = _sc_scatter_add(idxf, zaug, jnp.zeros((VQN, AW), jnp.float32))

    wpad = pl.pallas_call(
        _ema_body,
        in_specs=[
            _full((SC_NC, VQN, AW)), _full((VQN, VQD)), _full((VQN, 1)),
        ],
        out_specs=_full((VQN, AW)),
        out_shape=jax.ShapeDtypeStruct((VQN, AW), jnp.float32),
    )(parts, dict_val, dict_cnt.reshape(VQN, 1))

    wj = _sc_gather(wpad, idxf)

    recon, commit = pl.pallas_call(
        _decoder_body,
        grid=(B,),
        in_specs=[
            pl.BlockSpec((1, Tq, AW), lambda b: (b, 0, 0)),
            pl.BlockSpec((1, Tq, AW), lambda b: (b, 0, 0)),
            _full((3, VQD, 256)), _full((256,)), _full((256,)),
            _full((3, 256, 256)), _full((256,)), _full((256,)),
            _full((3, 256, 256)), _full((256,)), _full((256,)),
            _full((5, 256, Cx)), _full((Cx,)), _full((256, NGROUPS)),
        ],
        out_specs=[pl.BlockSpec((1, Cx, T), lambda b: (b, 0, 0)),
                   _full((1, 1))],
        out_shape=[jax.ShapeDtypeStruct((B, Cx, T), jnp.float32),
                   jax.ShapeDtypeStruct((1, 1), jnp.float32)],
    )(zaug.reshape(B, Tq, AW), wj.reshape(B, Tq, AW), wd2t, gd2, bd2,
      wd1at, gd1a, bd1a, wd1bt, gd1b, bd1b, wd0t, bd0, egrp)

    return recon, commit[0, 0]


# speedup vs baseline: 1.0001x; 1.0001x over previous
import functools

import jax
import jax.numpy as jnp
from jax.experimental import pallas as pl
from jax.experimental.pallas import tpu as pltpu, tpu_sc as plsc

VQN = 512
VQD = 64
NGROUPS = 16
NCARD = 8
EMA_ALPHA = 0.99
EPS = 1e-5


def _conv_rows(x, w_taps, k):
    T = x.shape[0]
    pad = k // 2
    xp = jnp.pad(x.astype(jnp.bfloat16), ((pad, pad), (0, 0)))
    acc = jax.lax.dot_general(
        xp[0:T, :], w_taps[0].astype(jnp.bfloat16),
        (((1,), (0,)), ((), ())), preferred_element_type=jnp.float32)
    for dt in range(1, k):
        acc = acc + jax.lax.dot_general(
            xp[dt:dt + T, :], w_taps[dt].astype(jnp.bfloat16),
            (((1,), (0,)), ((), ())), preferred_element_type=jnp.float32)
    return acc


def _conv_rows_im2col(x, w_taps, k):
    T, Cin = x.shape
    pad = k // 2
    xp = jnp.pad(x.astype(jnp.bfloat16), ((pad, pad), (0, 0)))
    xs = jnp.concatenate([xp[dt:dt + T, :] for dt in range(k)], axis=1)
    wf = w_taps[...].reshape(k * Cin, w_taps.shape[2]).astype(jnp.bfloat16)
    return jax.lax.dot_general(xs, wf, (((1,), (0,)), ((), ())),
                               preferred_element_type=jnp.float32)


def _gn_rows(h, g, b, egrp):
    T, C = h.shape
    n = (C // NGROUPS) * T
    hp = jax.lax.Precision.HIGHEST
    s_c = jnp.sum(h, axis=0, keepdims=True)
    s_g = jax.lax.dot_general(s_c, egrp, (((1,), (0,)), ((), ())),
                              precision=hp,
                              preferred_element_type=jnp.float32)
    m_c = jax.lax.dot_general(s_g, egrp, (((1,), (1,)), ((), ())),
                              precision=hp,
                              preferred_element_type=jnp.float32) / n
    d = h - m_c
    v_c = jnp.sum(d * d, axis=0, keepdims=True)
    v_g = jax.lax.dot_general(v_c, egrp, (((1,), (0,)), ((), ())),
                              precision=hp,
                              preferred_element_type=jnp.float32)
    v_cb = jax.lax.dot_general(v_g, egrp, (((1,), (1,)), ((), ())),
                               precision=hp,
                               preferred_element_type=jnp.float32) / n
    return d / jnp.sqrt(v_cb + EPS) * g[None, :] + b[None, :]


def _maxpool2_rows(h):
    T, C = h.shape
    return jnp.max(h.reshape(T // 2, 2, C), axis=1)


def _upsample2_rows(h):
    T, C = h.shape
    return jnp.broadcast_to(h[:, None, :], (T, 2, C)).reshape(2 * T, C)


def _encoder_body(x_ref, w0t_ref, g0_ref, b0_ref, wxd_ref, gx_ref, bx_ref,
                  w2t_ref, egrp_ref, out_ref):
    egrp = egrp_ref[...]
    g0 = g0_ref[...]
    b0 = b0_ref[...]
    gx = gx_ref[...]
    bx = bx_ref[...]

    for i in range(x_ref.shape[0]):
        xt = x_ref[i].T
        h = _conv_rows_im2col(xt, w0t_ref, 5)
        h = jax.nn.relu(_gn_rows(h, g0, b0, egrp))
        h = jax.nn.relu(h + _gn_rows(_conv_rows_im2col(h, wxd_ref, 3),
                                     gx, bx, egrp))
        h = _maxpool2_rows(h)
        h = jax.nn.relu(h + _gn_rows(_conv_rows_im2col(h, wxd_ref, 3),
                                     gx, bx, egrp))
        h = _maxpool2_rows(h)
        out_ref[i] = _conv_rows_im2col(h, w2t_ref, 3)


def _argmin_body(h2_ref, g2_ref, b2_ref, w_ref, zaug_ref, idx_ref):
    B, Tq, C = h2_ref.shape
    N = B * Tq
    Z0 = h2_ref[...].reshape(N, C)
    m = jnp.mean(Z0, axis=0, keepdims=True)
    d = Z0 - m
    v = jnp.mean(d * d, axis=0, keepdims=True)
    Z = d / jnp.sqrt(v + EPS) * g2_ref[...][None, :] + b2_ref[...][None, :]

    w = w_ref[...]
    hp = jax.lax.Precision.HIGHEST
    zz = jnp.sum(Z * Z, axis=1, keepdims=True)
    zw = jax.lax.dot_general(Z.astype(jnp.bfloat16), w.astype(jnp.bfloat16),
                             (((1,), (1,)), ((), ())),
                             preferred_element_type=jnp.float32)
    ww = jax.lax.dot_general(jnp.ones((1, C), jnp.float32), w * w,
                             (((1,), (1,)), ((), ())), precision=hp,
                             preferred_element_type=jnp.float32)
    dist = zz - 2.0 * zw + ww
    amin = jnp.min(dist, axis=1, keepdims=True)
    lane = jax.lax.broadcasted_iota(jnp.int32, (N, VQN), 1)
    idx = jnp.min(jnp.where(dist == amin, lane, jnp.int32(2 ** 30)),
                  axis=1, keepdims=True)
    idx_ref[...] = idx

    alane = jax.lax.broadcasted_iota(jnp.int32, (N, AW), 1)
    zaug_ref[...] = (jnp.pad(Z, ((0, 0), (0, AW - C)))
                     + (alane == C).astype(jnp.float32))


def _ema_body(parts_ref, w_ref, c_ref, wpad_ref):
    acc = parts_ref[0] + parts_ref[1]
    counts = acc[:, VQD:VQD + 1]
    sums = acc[:, :VQD]
    w = w_ref[...]
    c = c_ref[...]
    mask = counts > 0
    c_new = jnp.where(mask, EMA_ALPHA * c + (1.0 - EMA_ALPHA) * counts, c)
    w_new = jnp.where(
        mask,
        EMA_ALPHA * w + (1.0 - EMA_ALPHA) * sums / jnp.maximum(c_new, 1e-12),
        w)
    wpad_ref[...] = jnp.pad(w_new, ((0, 0), (0, AW - VQD)))


NTOK = 4096
AW = 128
SPP = 2
SC_NC = 2
SC_NS = 16
SC_NW = SC_NC * SC_NS
SC_RPW = NTOK // SC_NW
SC_CPS = VQN // SC_NS

_sc_mesh = plsc.VectorSubcoreMesh(core_axis_name="c", subcore_axis_name="s")


@functools.partial(
    pl.kernel, mesh=_sc_mesh,
    out_type=jax.ShapeDtypeStruct((SC_NC, VQN, AW), jnp.float32),
    scratch_types=[
        pltpu.VMEM((SC_RPW,), jnp.int32),
        pltpu.VMEM((SC_RPW, AW), jnp.float32),
        pltpu.VMEM((SC_CPS, AW), jnp.float32),
        pltpu.VMEM_SHARED((VQN, AW), jnp.float32),
        pltpu.SemaphoreType.DMA,
    ],
)
def _sc_scatter_add(idx_hbm, zaug_hbm, zero_hbm, out_hbm,
                    idx_v, zaug_v, acc_v, acc_sh, sem):
    cid = jax.lax.axis_index("c")
    sid = jax.lax.axis_index("s")
    wid = sid * SC_NC + cid

    pltpu.sync_copy(zero_hbm.at[pl.ds(sid * SC_CPS, SC_CPS)],
                    acc_sh.at[pl.ds(sid * SC_CPS, SC_CPS)])
    plsc.subcore_barrier()

    base = wid * SC_RPW
    pltpu.sync_copy(idx_hbm.at[pl.ds(base, SC_RPW)], idx_v)
    pltpu.sync_copy(zaug_hbm.at[pl.ds(base, SC_RPW)], zaug_v)
    pltpu.sync_copy(zaug_v, acc_sh.at[idx_v], add=True)
    plsc.subcore_barrier()

    pltpu.sync_copy(acc_sh.at[pl.ds(sid * SC_CPS, SC_CPS)], acc_v)
    pltpu.sync_copy(acc_v, out_hbm.at[cid].at[pl.ds(sid * SC_CPS, SC_CPS)])


@functools.partial(
    pl.kernel, mesh=_sc_mesh,
    out_type=jax.ShapeDtypeStruct((NTOK, AW), jnp.float32),
    scratch_types=[
        pltpu.VMEM((SC_RPW,), jnp.int32),
        pltpu.VMEM((SC_RPW, AW), jnp.float32),
        pltpu.SemaphoreType.DMA,
    ],
)
def _sc_gather(table_hbm, idx_hbm, out_hbm, idx_v, rows_v, sem):
    cid = jax.lax.axis_index("c")
    sid = jax.lax.axis_index("s")
    wid = sid * SC_NC + cid
    base = wid * SC_RPW
    pltpu.sync_copy(idx_hbm.at[pl.ds(base, SC_RPW)], idx_v)
    pltpu.async_copy(table_hbm.at[idx_v], rows_v, sem).wait()
    pltpu.sync_copy(rows_v, out_hbm.at[pl.ds(base, SC_RPW)])


def _decoder_body(zaug_ref, wj_ref, wd2t_ref, gd2_ref, bd2_ref, wd1at_ref,
                  gd1a_ref, bd1a_ref, wd1bt_ref, gd1b_ref, bd1b_ref,
                  wd0t_ref, bd0_ref, egrp_ref, out_ref, commit_ref):
    egrp = egrp_ref[...]
    gd2 = gd2_ref[...]
    bd2 = bd2_ref[...]
    gd1a = gd1a_ref[...]
    bd1a = bd1a_ref[...]
    gd1b = gd1b_ref[...]
    bd1b = bd1b_ref[...]

    def deres1(t):
        o = jax.nn.relu(_gn_rows(_conv_rows(t, wd1at_ref, 3), gd1a, bd1a, egrp))
        o = _gn_rows(_conv_rows(o, wd1bt_ref, 3), gd1b, bd1b, egrp)
        return jax.nn.relu(t + o)

    b = pl.program_id(0)
    nb = pl.num_programs(0)

    @pl.when(b == 0)
    def _():
        commit_ref[...] = jnp.zeros((1, 1), jnp.float32)

    part = jnp.zeros((), jnp.float32)
    for i in range(zaug_ref.shape[0]):
        z = zaug_ref[i][:, :VQD]
        hq = wj_ref[i][:, :VQD]
        dz = z - hq
        part = part + jnp.sum(dz * dz)

        t = jax.nn.relu(_gn_rows(_conv_rows(hq, wd2t_ref, 3),
                                 gd2, bd2, egrp))
        t = _upsample2_rows(t)
        t = deres1(t)
        t = _upsample2_rows(t)
        t = deres1(t)
        r = _conv_rows(t, wd0t_ref, 5) + bd0_ref[...][None, :]
        out_ref[i] = r.T

    commit_ref[...] += part.reshape(1, 1)

    @pl.when(b == nb - 1)
    def _():
        commit_ref[...] = commit_ref[...] / NTOK


def _full(bspec):
    return pl.BlockSpec(bspec, lambda *_: (0,) * len(bspec))


def _block_diag_taps(wx):
    co, ci_g, k = wx.shape
    gsz = ci_g
    ngr = co // gsz
    wt = jnp.transpose(wx, (2, 0, 1))
    out = jnp.zeros((k, co, co), jnp.float32)
    for g in range(ngr):
        blk = jnp.transpose(wt[:, g * gsz:(g + 1) * gsz, :], (0, 2, 1))
        out = jax.lax.dynamic_update_slice(out, blk, (0, g * gsz, g * gsz))
    return out


def kernel(x, W0, g0, b0, Wx, gx, bx, W2, g2, b2, Wd2, gd2, bd2, Wd1a, gd1a,
           bd1a, Wd1b, gd1b, bd1b, Wd0, bd0, dict_val, dict_cnt):
    B, Cx, T = x.shape
    Tq = T // 4

    w0t = jnp.transpose(W0, (2, 1, 0))
    wxd = _block_diag_taps(Wx)
    w2t = jnp.transpose(W2, (2, 1, 0))
    wd2t = jnp.transpose(Wd2, (2, 0, 1))
    wd1at = jnp.transpose(Wd1a, (2, 0, 1))
    wd1bt = jnp.transpose(Wd1b, (2, 0, 1))
    wd0t = jnp.transpose(Wd0, (2, 0, 1))
    ch = jax.lax.broadcasted_iota(jnp.int32, (256, NGROUPS), 0)
    gr = jax.lax.broadcasted_iota(jnp.int32, (256, NGROUPS), 1)
    egrp = (ch // (256 // NGROUPS) == gr).astype(jnp.float32)

    h2 = pl.pallas_call(
        _encoder_body,
        grid=(B // SPP,),
        in_specs=[
            pl.BlockSpec((SPP, Cx, T), lambda b: (b, 0, 0)),
            _full((5, Cx, 256)), _full((256,)), _full((256,)),
            _full((3, 256, 256)), _full((256,)), _full((256,)),
            _full((3, 256, VQD)), _full((256, NGROUPS)),
        ],
        out_specs=pl.BlockSpec((SPP, Tq, VQD), lambda b: (b, 0, 0)),
        out_shape=jax.ShapeDtypeStruct((B, Tq, VQD), jnp.float32),
    )(x, w0t, g0, b0, wxd, gx, bx, w2t, egrp)

    zaug, idxm = pl.pallas_call(
        _argmin_body,
        in_specs=[
            _full((B, Tq, VQD)), _full((VQD,)), _full((VQD,)),
            _full((VQN, VQD)),
        ],
        out_specs=[_full((NTOK, AW)), _full((NTOK, 1))],
        out_shape=[
            jax.ShapeDtypeStruct((NTOK, AW), jnp.float32),
            jax.ShapeDtypeStruct((NTOK, 1), jnp.int32),
        ],
    )(h2, g2, b2, dict_val)

    idxf = idxm.reshape(NTOK)
    parts = _sc_scatter_add(idxf, zaug, jnp.zeros((VQN, AW), jnp.float32))

    wpad = pl.pallas_call(
        _ema_body,
        in_specs=[
            _full((SC_NC, VQN, AW)), _full((VQN, VQD)), _full((VQN, 1)),
        ],
        out_specs=_full((VQN, AW)),
        out_shape=jax.ShapeDtypeStruct((VQN, AW), jnp.float32),
    )(parts, dict_val, dict_cnt.reshape(VQN, 1))

    wj = _sc_gather(wpad, idxf)

    recon, commit = pl.pallas_call(
        _decoder_body,
        grid=(B // SPP,),
        in_specs=[
            pl.BlockSpec((SPP, Tq, AW), lambda b: (b, 0, 0)),
            pl.BlockSpec((SPP, Tq, AW), lambda b: (b, 0, 0)),
            _full((3, VQD, 256)), _full((256,)), _full((256,)),
            _full((3, 256, 256)), _full((256,)), _full((256,)),
            _full((3, 256, 256)), _full((256,)), _full((256,)),
            _full((5, 256, Cx)), _full((Cx,)), _full((256, NGROUPS)),
        ],
        out_specs=[pl.BlockSpec((SPP, Cx, T), lambda b: (b, 0, 0)),
                   _full((1, 1))],
        out_shape=[jax.ShapeDtypeStruct((B, Cx, T), jnp.float32),
                   jax.ShapeDtypeStruct((1, 1), jnp.float32)],
    )(zaug.reshape(B, Tq, AW), wj.reshape(B, Tq, AW), wd2t, gd2, bd2,
      wd1at, gd1a, bd1a, wd1bt, gd1b, bd1b, wd0t, bd0, egrp)

    return recon, commit[0, 0]


# speedup vs baseline: 1.0390x; 1.0390x over previous
import functools

import jax
import jax.numpy as jnp
from jax.experimental import pallas as pl
from jax.experimental.pallas import tpu as pltpu, tpu_sc as plsc

VQN = 512
VQD = 64
NGROUPS = 16
NCARD = 8
EMA_ALPHA = 0.99
EPS = 1e-5


def _conv_rows(x, w_taps, k):
    T = x.shape[0]
    pad = k // 2
    xp = jnp.pad(x.astype(jnp.bfloat16), ((pad, pad), (0, 0)))
    acc = jax.lax.dot_general(
        xp[0:T, :], w_taps[0].astype(jnp.bfloat16),
        (((1,), (0,)), ((), ())), preferred_element_type=jnp.float32)
    for dt in range(1, k):
        acc = acc + jax.lax.dot_general(
            xp[dt:dt + T, :], w_taps[dt].astype(jnp.bfloat16),
            (((1,), (0,)), ((), ())), preferred_element_type=jnp.float32)
    return acc


def _conv_rows_im2col(x, w_taps, k):
    T, Cin = x.shape
    pad = k // 2
    xp = jnp.pad(x.astype(jnp.bfloat16), ((pad, pad), (0, 0)))
    xs = jnp.concatenate([xp[dt:dt + T, :] for dt in range(k)], axis=1)
    wf = w_taps[...].reshape(k * Cin, w_taps.shape[2]).astype(jnp.bfloat16)
    return jax.lax.dot_general(xs, wf, (((1,), (0,)), ((), ())),
                               preferred_element_type=jnp.float32)


def _gn_rows(h, g, b, egrp):
    T, C = h.shape
    n = (C // NGROUPS) * T
    hp = jax.lax.Precision.HIGHEST
    s_c = jnp.sum(h, axis=0, keepdims=True)
    s_g = jax.lax.dot_general(s_c, egrp, (((1,), (0,)), ((), ())),
                              precision=hp,
                              preferred_element_type=jnp.float32)
    m_c = jax.lax.dot_general(s_g, egrp, (((1,), (1,)), ((), ())),
                              precision=hp,
                              preferred_element_type=jnp.float32) / n
    d = h - m_c
    v_c = jnp.sum(d * d, axis=0, keepdims=True)
    v_g = jax.lax.dot_general(v_c, egrp, (((1,), (0,)), ((), ())),
                              precision=hp,
                              preferred_element_type=jnp.float32)
    v_cb = jax.lax.dot_general(v_g, egrp, (((1,), (1,)), ((), ())),
                               precision=hp,
                               preferred_element_type=jnp.float32) / n
    return d / jnp.sqrt(v_cb + EPS) * g[None, :] + b[None, :]


def _maxpool2_rows(h):
    T, C = h.shape
    return jnp.max(h.reshape(T // 2, 2, C), axis=1)


def _upsample2_rows(h):
    T, C = h.shape
    return jnp.broadcast_to(h[:, None, :], (T, 2, C)).reshape(2 * T, C)


def _encoder_body(x_ref, w0t_ref, g0_ref, b0_ref, wxd_ref, gx_ref, bx_ref,
                  w2t_ref, egrp_ref, out_ref):
    egrp = egrp_ref[...]
    g0 = g0_ref[...]
    b0 = b0_ref[...]
    gx = gx_ref[...]
    bx = bx_ref[...]

    for i in range(x_ref.shape[0]):
        xt = x_ref[i].T
        h = _conv_rows_im2col(xt, w0t_ref, 5)
        h = jax.nn.relu(_gn_rows(h, g0, b0, egrp))
        h = jax.nn.relu(h + _gn_rows(_conv_rows_im2col(h, wxd_ref, 3),
                                     gx, bx, egrp))
        h = _maxpool2_rows(h)
        h = jax.nn.relu(h + _gn_rows(_conv_rows_im2col(h, wxd_ref, 3),
                                     gx, bx, egrp))
        h = _maxpool2_rows(h)
        out_ref[i] = _conv_rows_im2col(h, w2t_ref, 3)


def _argmin_body(h2_ref, g2_ref, b2_ref, w_ref, zaug_ref, idx_ref):
    B, Tq, C = h2_ref.shape
    N = B * Tq
    Z0 = h2_ref[...].reshape(N, C)
    m = jnp.mean(Z0, axis=0, keepdims=True)
    d = Z0 - m
    v = jnp.mean(d * d, axis=0, keepdims=True)
    Z = d / jnp.sqrt(v + EPS) * g2_ref[...][None, :] + b2_ref[...][None, :]

    w = w_ref[...]
    hp = jax.lax.Precision.HIGHEST
    zz = jnp.sum(Z * Z, axis=1, keepdims=True)
    zw = jax.lax.dot_general(Z.astype(jnp.bfloat16), w.astype(jnp.bfloat16),
                             (((1,), (1,)), ((), ())),
                             preferred_element_type=jnp.float32)
    ww = jax.lax.dot_general(jnp.ones((1, C), jnp.float32), w * w,
                             (((1,), (1,)), ((), ())), precision=hp,
                             preferred_element_type=jnp.float32)
    dist = zz - 2.0 * zw + ww
    amin = jnp.min(dist, axis=1, keepdims=True)
    lane = jax.lax.broadcasted_iota(jnp.int32, (N, VQN), 1)
    idx = jnp.min(jnp.where(dist == amin, lane, jnp.int32(2 ** 30)),
                  axis=1, keepdims=True)
    idx_ref[...] = idx

    alane = jax.lax.broadcasted_iota(jnp.int32, (N, AW), 1)
    zaug_ref[...] = (jnp.pad(Z, ((0, 0), (0, AW - C)))
                     + (alane == C).astype(jnp.float32))


def _ema_body(parts_ref, w_ref, c_ref, wpad_ref):
    acc = parts_ref[0] + parts_ref[1]
    counts = acc[:, VQD:VQD + 1]
    sums = acc[:, :VQD]
    w = w_ref[...]
    c = c_ref[...]
    mask = counts > 0
    c_new = jnp.where(mask, EMA_ALPHA * c + (1.0 - EMA_ALPHA) * counts, c)
    w_new = jnp.where(
        mask,
        EMA_ALPHA * w + (1.0 - EMA_ALPHA) * sums / jnp.maximum(c_new, 1e-12),
        w)
    wpad_ref[...] = jnp.pad(w_new, ((0, 0), (0, AW - VQD)))


NTOK = 4096
AW = 128
SPP = 2
SC_NC = 2
SC_NS = 16
SC_NW = SC_NC * SC_NS
SC_RPW = NTOK // SC_NW
SC_CPS = VQN // SC_NS

_sc_mesh = plsc.VectorSubcoreMesh(core_axis_name="c", subcore_axis_name="s")


@functools.partial(
    pl.kernel, mesh=_sc_mesh,
    out_type=jax.ShapeDtypeStruct((SC_NC, VQN, AW), jnp.float32),
    scratch_types=[
        pltpu.VMEM((SC_RPW,), jnp.int32),
        pltpu.VMEM((SC_RPW, AW), jnp.float32),
        pltpu.VMEM((SC_CPS, AW), jnp.float32),
        pltpu.VMEM_SHARED((VQN, AW), jnp.float32),
        pltpu.SemaphoreType.DMA,
    ],
)
def _sc_scatter_add(idx_hbm, zaug_hbm, zero_hbm, out_hbm,
                    idx_v, zaug_v, acc_v, acc_sh, sem):
    cid = jax.lax.axis_index("c")
    sid = jax.lax.axis_index("s")
    wid = sid * SC_NC + cid

    pltpu.sync_copy(zero_hbm.at[pl.ds(sid * SC_CPS, SC_CPS)],
                    acc_sh.at[pl.ds(sid * SC_CPS, SC_CPS)])
    plsc.subcore_barrier()

    base = wid * SC_RPW
    pltpu.sync_copy(idx_hbm.at[pl.ds(base, SC_RPW)], idx_v)
    pltpu.sync_copy(zaug_hbm.at[pl.ds(base, SC_RPW)], zaug_v)
    pltpu.sync_copy(zaug_v, acc_sh.at[idx_v], add=True)
    plsc.subcore_barrier()

    pltpu.sync_copy(acc_sh.at[pl.ds(sid * SC_CPS, SC_CPS)], acc_v)
    pltpu.sync_copy(acc_v, out_hbm.at[cid].at[pl.ds(sid * SC_CPS, SC_CPS)])


@functools.partial(
    pl.kernel, mesh=_sc_mesh,
    out_type=jax.ShapeDtypeStruct((NTOK, AW), jnp.float32),
    scratch_types=[
        pltpu.VMEM((SC_RPW,), jnp.int32),
        pltpu.VMEM((SC_RPW, AW), jnp.float32),
        pltpu.SemaphoreType.DMA,
    ],
)
def _sc_gather(table_hbm, idx_hbm, out_hbm, idx_v, rows_v, sem):
    cid = jax.lax.axis_index("c")
    sid = jax.lax.axis_index("s")
    wid = sid * SC_NC + cid
    base = wid * SC_RPW
    pltpu.sync_copy(idx_hbm.at[pl.ds(base, SC_RPW)], idx_v)
    pltpu.async_copy(table_hbm.at[idx_v], rows_v, sem).wait()
    pltpu.sync_copy(rows_v, out_hbm.at[pl.ds(base, SC_RPW)])


def _decoder_body(zaug_ref, wj_ref, wd2t_ref, gd2_ref, bd2_ref, wd1at_ref,
                  gd1a_ref, bd1a_ref, wd1bt_ref, gd1b_ref, bd1b_ref,
                  wd0t_ref, bd0_ref, egrp_ref, out_ref, commit_ref):
    egrp = egrp_ref[...]
    gd2 = gd2_ref[...]
    bd2 = bd2_ref[...]
    gd1a = gd1a_ref[...]
    bd1a = bd1a_ref[...]
    gd1b = gd1b_ref[...]
    bd1b = bd1b_ref[...]

    def deres1(t):
        o = jax.nn.relu(_gn_rows(_conv_rows_im2col(t, wd1at_ref, 3),
                                 gd1a, bd1a, egrp))
        o = _gn_rows(_conv_rows_im2col(o, wd1bt_ref, 3), gd1b, bd1b, egrp)
        return jax.nn.relu(t + o)

    b = pl.program_id(0)
    nb = pl.num_programs(0)

    @pl.when(b == 0)
    def _():
        commit_ref[...] = jnp.zeros((1, 1), jnp.float32)

    part = jnp.zeros((), jnp.float32)
    for i in range(zaug_ref.shape[0]):
        z = zaug_ref[i][:, :VQD]
        hq = wj_ref[i][:, :VQD]
        dz = z - hq
        part = part + jnp.sum(dz * dz)

        t = jax.nn.relu(_gn_rows(_conv_rows_im2col(hq, wd2t_ref, 3),
                                 gd2, bd2, egrp))
        t = _upsample2_rows(t)
        t = deres1(t)
        t = _upsample2_rows(t)
        t = deres1(t)
        r = (_conv_rows_im2col(t, wd0t_ref, 5)
             + bd0_ref[...][None, :])
        out_ref[i] = r.T

    commit_ref[...] += part.reshape(1, 1)

    @pl.when(b == nb - 1)
    def _():
        commit_ref[...] = commit_ref[...] / NTOK


def _full(bspec):
    return pl.BlockSpec(bspec, lambda *_: (0,) * len(bspec))


def _block_diag_taps(wx):
    co, ci_g, k = wx.shape
    gsz = ci_g
    ngr = co // gsz
    wt = jnp.transpose(wx, (2, 0, 1))
    out = jnp.zeros((k, co, co), jnp.float32)
    for g in range(ngr):
        blk = jnp.transpose(wt[:, g * gsz:(g + 1) * gsz, :], (0, 2, 1))
        out = jax.lax.dynamic_update_slice(out, blk, (0, g * gsz, g * gsz))
    return out


def kernel(x, W0, g0, b0, Wx, gx, bx, W2, g2, b2, Wd2, gd2, bd2, Wd1a, gd1a,
           bd1a, Wd1b, gd1b, bd1b, Wd0, bd0, dict_val, dict_cnt):
    B, Cx, T = x.shape
    Tq = T // 4

    w0t = jnp.transpose(W0, (2, 1, 0))
    wxd = _block_diag_taps(Wx)
    w2t = jnp.transpose(W2, (2, 1, 0))
    wd2t = jnp.transpose(Wd2, (2, 0, 1))
    wd1at = jnp.transpose(Wd1a, (2, 0, 1))
    wd1bt = jnp.transpose(Wd1b, (2, 0, 1))
    wd0t = jnp.transpose(Wd0, (2, 0, 1))
    ch = jax.lax.broadcasted_iota(jnp.int32, (256, NGROUPS), 0)
    gr = jax.lax.broadcasted_iota(jnp.int32, (256, NGROUPS), 1)
    egrp = (ch // (256 // NGROUPS) == gr).astype(jnp.float32)

    h2 = pl.pallas_call(
        _encoder_body,
        grid=(B // SPP,),
        in_specs=[
            pl.BlockSpec((SPP, Cx, T), lambda b: (b, 0, 0)),
            _full((5, Cx, 256)), _full((256,)), _full((256,)),
            _full((3, 256, 256)), _full((256,)), _full((256,)),
            _full((3, 256, VQD)), _full((256, NGROUPS)),
        ],
        out_specs=pl.BlockSpec((SPP, Tq, VQD), lambda b: (b, 0, 0)),
        out_shape=jax.ShapeDtypeStruct((B, Tq, VQD), jnp.float32),
    )(x, w0t, g0, b0, wxd, gx, bx, w2t, egrp)

    zaug, idxm = pl.pallas_call(
        _argmin_body,
        in_specs=[
            _full((B, Tq, VQD)), _full((VQD,)), _full((VQD,)),
            _full((VQN, VQD)),
        ],
        out_specs=[_full((NTOK, AW)), _full((NTOK, 1))],
        out_shape=[
            jax.ShapeDtypeStruct((NTOK, AW), jnp.float32),
            jax.ShapeDtypeStruct((NTOK, 1), jnp.int32),
        ],
    )(h2, g2, b2, dict_val)

    idxf = idxm.reshape(NTOK)
    parts = _sc_scatter_add(idxf, zaug, jnp.zeros((VQN, AW), jnp.float32))

    wpad = pl.pallas_call(
        _ema_body,
        in_specs=[
            _full((SC_NC, VQN, AW)), _full((VQN, VQD)), _full((VQN, 1)),
        ],
        out_specs=_full((VQN, AW)),
        out_shape=jax.ShapeDtypeStruct((VQN, AW), jnp.float32),
    )(parts, dict_val, dict_cnt.reshape(VQN, 1))

    wj = _sc_gather(wpad, idxf)

    recon, commit = pl.pallas_call(
        _decoder_body,
        grid=(B // SPP,),
        in_specs=[
            pl.BlockSpec((SPP, Tq, AW), lambda b: (b, 0, 0)),
            pl.BlockSpec((SPP, Tq, AW), lambda b: (b, 0, 0)),
            _full((3, VQD, 256)), _full((256,)), _full((256,)),
            _full((3, 256, 256)), _full((256,)), _full((256,)),
            _full((3, 256, 256)), _full((256,)), _full((256,)),
            _full((5, 256, Cx)), _full((Cx,)), _full((256, NGROUPS)),
        ],
        out_specs=[pl.BlockSpec((SPP, Cx, T), lambda b: (b, 0, 0)),
                   _full((1, 1))],
        out_shape=[jax.ShapeDtypeStruct((B, Cx, T), jnp.float32),
                   jax.ShapeDtypeStruct((1, 1), jnp.float32)],
    )(zaug.reshape(B, Tq, AW), wj.reshape(B, Tq, AW), wd2t, gd2, bd2,
      wd1at, gd1a, bd1a, wd1bt, gd1b, bd1b, wd0t, bd0, egrp)

    return recon, commit[0, 0]


# speedup vs baseline: 1.2129x; 1.1674x over previous
import functools

import jax
import jax.numpy as jnp
from jax.experimental import pallas as pl
from jax.experimental.pallas import tpu as pltpu, tpu_sc as plsc

VQN = 512
VQD = 64
NGROUPS = 16
NCARD = 8
EMA_ALPHA = 0.99
EPS = 1e-5


def _conv_rows(x, w_taps, k):
    T = x.shape[0]
    pad = k // 2
    xp = jnp.pad(x.astype(jnp.bfloat16), ((pad, pad), (0, 0)))
    acc = jax.lax.dot_general(
        xp[0:T, :], w_taps[0].astype(jnp.bfloat16),
        (((1,), (0,)), ((), ())), preferred_element_type=jnp.float32)
    for dt in range(1, k):
        acc = acc + jax.lax.dot_general(
            xp[dt:dt + T, :], w_taps[dt].astype(jnp.bfloat16),
            (((1,), (0,)), ((), ())), preferred_element_type=jnp.float32)
    return acc


def _conv_rows_im2col(x, w_taps, k):
    T, Cin = x.shape
    pad = k // 2
    xp = jnp.pad(x.astype(jnp.bfloat16), ((pad, pad), (0, 0)))
    xs = jnp.concatenate([xp[dt:dt + T, :] for dt in range(k)], axis=1)
    wf = w_taps[...].reshape(k * Cin, w_taps.shape[2]).astype(jnp.bfloat16)
    return jax.lax.dot_general(xs, wf, (((1,), (0,)), ((), ())),
                               preferred_element_type=jnp.float32)


def _gn_rows(h, g, b, egrp):
    T, C = h.shape
    hp = jax.lax.Precision.HIGHEST
    ht = h.T
    xg = ht.reshape(NGROUPS, C // NGROUPS, T)
    m3 = jnp.mean(xg, axis=(1, 2), keepdims=True)
    v3 = jnp.mean((xg - m3) ** 2, axis=(1, 2), keepdims=True)
    m_row = m3.reshape(NGROUPS, 1).T
    v_row = v3.reshape(NGROUPS, 1).T
    m_c = jax.lax.dot_general(m_row, egrp, (((1,), (1,)), ((), ())),
                              precision=hp,
                              preferred_element_type=jnp.float32)
    v_c = jax.lax.dot_general(v_row, egrp, (((1,), (1,)), ((), ())),
                              precision=hp,
                              preferred_element_type=jnp.float32)
    return (h - m_c) / jnp.sqrt(v_c + EPS) * g[None, :] + b[None, :]


def _maxpool2_rows(h):
    T, C = h.shape
    return jnp.max(h.reshape(T // 2, 2, C), axis=1)


def _upsample2_rows(h):
    T, C = h.shape
    return jnp.broadcast_to(h[:, None, :], (T, 2, C)).reshape(2 * T, C)


def _encoder_body(x_ref, w0t_ref, g0_ref, b0_ref, wxd_ref, gx_ref, bx_ref,
                  w2t_ref, egrp_ref, out_ref):
    egrp = egrp_ref[...]
    g0 = g0_ref[...]
    b0 = b0_ref[...]
    gx = gx_ref[...]
    bx = bx_ref[...]

    for i in range(x_ref.shape[0]):
        xt = x_ref[i].T
        h = _conv_rows_im2col(xt, w0t_ref, 5)
        h = jax.nn.relu(_gn_rows(h, g0, b0, egrp))
        h = jax.nn.relu(h + _gn_rows(_conv_rows_im2col(h, wxd_ref, 3),
                                     gx, bx, egrp))
        h = _maxpool2_rows(h)
        h = jax.nn.relu(h + _gn_rows(_conv_rows_im2col(h, wxd_ref, 3),
                                     gx, bx, egrp))
        h = _maxpool2_rows(h)
        out_ref[i] = _conv_rows_im2col(h, w2t_ref, 3)


def _argmin_body(h2_ref, g2_ref, b2_ref, w_ref, zaug_ref, idx_ref):
    B, Tq, C = h2_ref.shape
    N = B * Tq
    h2 = h2_ref[...]
    h2t = jnp.transpose(h2, (0, 2, 1))
    m3 = jnp.mean(h2t, axis=(0, 2), keepdims=True)
    v3 = jnp.mean((h2t - m3) ** 2, axis=(0, 2), keepdims=True)
    m = m3.reshape(C, 1).T
    v = v3.reshape(C, 1).T
    Z0 = h2.reshape(N, C)
    Z = ((Z0 - m) / jnp.sqrt(v + EPS) * g2_ref[...][None, :]
         + b2_ref[...][None, :])

    w = w_ref[...]
    hp = jax.lax.Precision.HIGHEST
    zz = jnp.sum(Z * Z, axis=1, keepdims=True)
    zw = jax.lax.dot_general(Z.astype(jnp.bfloat16), w.astype(jnp.bfloat16),
                             (((1,), (1,)), ((), ())),
                             preferred_element_type=jnp.float32)
    ww = jax.lax.dot_general(jnp.ones((1, C), jnp.float32), w * w,
                             (((1,), (1,)), ((), ())), precision=hp,
                             preferred_element_type=jnp.float32)
    dist = zz - 2.0 * zw + ww
    amin = jnp.min(dist, axis=1, keepdims=True)
    lane = jax.lax.broadcasted_iota(jnp.int32, (N, VQN), 1)
    idx = jnp.min(jnp.where(dist == amin, lane, jnp.int32(2 ** 30)),
                  axis=1, keepdims=True)
    idx_ref[...] = idx

    alane = jax.lax.broadcasted_iota(jnp.int32, (N, AW), 1)
    zaug_ref[...] = (jnp.pad(Z, ((0, 0), (0, AW - C)))
                     + (alane == C).astype(jnp.float32))


def _ema_body(parts_ref, w_ref, c_ref, wpad_ref):
    acc = parts_ref[0] + parts_ref[1]
    counts = acc[:, VQD:VQD + 1]
    sums = acc[:, :VQD]
    w = w_ref[...]
    c = c_ref[...]
    mask = counts > 0
    c_new = jnp.where(mask, EMA_ALPHA * c + (1.0 - EMA_ALPHA) * counts, c)
    w_new = jnp.where(
        mask,
        EMA_ALPHA * w + (1.0 - EMA_ALPHA) * sums / jnp.maximum(c_new, 1e-12),
        w)
    wpad_ref[...] = jnp.pad(w_new, ((0, 0), (0, AW - VQD)))


NTOK = 4096
AW = 128
SPP = 2
SC_NC = 2
SC_NS = 16
SC_NW = SC_NC * SC_NS
SC_RPW = NTOK // SC_NW
SC_CPS = VQN // SC_NS

_sc_mesh = plsc.VectorSubcoreMesh(core_axis_name="c", subcore_axis_name="s")


@functools.partial(
    pl.kernel, mesh=_sc_mesh,
    out_type=jax.ShapeDtypeStruct((SC_NC, VQN, AW), jnp.float32),
    scratch_types=[
        pltpu.VMEM((SC_RPW,), jnp.int32),
        pltpu.VMEM((SC_RPW, AW), jnp.float32),
        pltpu.VMEM((SC_CPS, AW), jnp.float32),
        pltpu.VMEM_SHARED((VQN, AW), jnp.float32),
        pltpu.SemaphoreType.DMA,
    ],
)
def _sc_scatter_add(idx_hbm, zaug_hbm, zero_hbm, out_hbm,
                    idx_v, zaug_v, acc_v, acc_sh, sem):
    cid = jax.lax.axis_index("c")
    sid = jax.lax.axis_index("s")
    wid = sid * SC_NC + cid

    pltpu.sync_copy(zero_hbm.at[pl.ds(sid * SC_CPS, SC_CPS)],
                    acc_sh.at[pl.ds(sid * SC_CPS, SC_CPS)])
    plsc.subcore_barrier()

    base = wid * SC_RPW
    pltpu.sync_copy(idx_hbm.at[pl.ds(base, SC_RPW)], idx_v)
    pltpu.sync_copy(zaug_hbm.at[pl.ds(base, SC_RPW)], zaug_v)
    pltpu.sync_copy(zaug_v, acc_sh.at[idx_v], add=True)
    plsc.subcore_barrier()

    pltpu.sync_copy(acc_sh.at[pl.ds(sid * SC_CPS, SC_CPS)], acc_v)
    pltpu.sync_copy(acc_v, out_hbm.at[cid].at[pl.ds(sid * SC_CPS, SC_CPS)])


@functools.partial(
    pl.kernel, mesh=_sc_mesh,
    out_type=jax.ShapeDtypeStruct((NTOK, AW), jnp.float32),
    scratch_types=[
        pltpu.VMEM((SC_RPW,), jnp.int32),
        pltpu.VMEM((SC_RPW, AW), jnp.float32),
        pltpu.SemaphoreType.DMA,
    ],
)
def _sc_gather(table_hbm, idx_hbm, out_hbm, idx_v, rows_v, sem):
    cid = jax.lax.axis_index("c")
    sid = jax.lax.axis_index("s")
    wid = sid * SC_NC + cid
    base = wid * SC_RPW
    pltpu.sync_copy(idx_hbm.at[pl.ds(base, SC_RPW)], idx_v)
    pltpu.async_copy(table_hbm.at[idx_v], rows_v, sem).wait()
    pltpu.sync_copy(rows_v, out_hbm.at[pl.ds(base, SC_RPW)])


def _decoder_body(zaug_ref, wj_ref, wd2t_ref, gd2_ref, bd2_ref, wd1at_ref,
                  gd1a_ref, bd1a_ref, wd1bt_ref, gd1b_ref, bd1b_ref,
                  wd0t_ref, bd0_ref, egrp_ref, out_ref, commit_ref):
    egrp = egrp_ref[...]
    gd2 = gd2_ref[...]
    bd2 = bd2_ref[...]
    gd1a = gd1a_ref[...]
    bd1a = bd1a_ref[...]
    gd1b = gd1b_ref[...]
    bd1b = bd1b_ref[...]

    def deres1(t):
        o = jax.nn.relu(_gn_rows(_conv_rows_im2col(t, wd1at_ref, 3),
                                 gd1a, bd1a, egrp))
        o = _gn_rows(_conv_rows_im2col(o, wd1bt_ref, 3), gd1b, bd1b, egrp)
        return jax.nn.relu(t + o)

    b = pl.program_id(0)
    nb = pl.num_programs(0)

    @pl.when(b == 0)
    def _():
        commit_ref[...] = jnp.zeros((1, 1), jnp.float32)

    part = jnp.zeros((), jnp.float32)
    for i in range(zaug_ref.shape[0]):
        z = zaug_ref[i][:, :VQD]
        hq = wj_ref[i][:, :VQD]
        dz = z - hq
        part = part + jnp.sum(dz * dz)

        t = jax.nn.relu(_gn_rows(_conv_rows_im2col(hq, wd2t_ref, 3),
                                 gd2, bd2, egrp))
        t = _upsample2_rows(t)
        t = deres1(t)
        t = _upsample2_rows(t)
        t = deres1(t)
        r = (_conv_rows_im2col(t, wd0t_ref, 5)
             + bd0_ref[...][None, :])
        out_ref[i] = r.T

    commit_ref[...] += part.reshape(1, 1)

    @pl.when(b == nb - 1)
    def _():
        commit_ref[...] = commit_ref[...] / NTOK


def _full(bspec):
    return pl.BlockSpec(bspec, lambda *_: (0,) * len(bspec))


def _block_diag_taps(wx):
    co, ci_g, k = wx.shape
    gsz = ci_g
    ngr = co // gsz
    wt = jnp.transpose(wx, (2, 0, 1))
    out = jnp.zeros((k, co, co), jnp.float32)
    for g in range(ngr):
        blk = jnp.transpose(wt[:, g * gsz:(g + 1) * gsz, :], (0, 2, 1))
        out = jax.lax.dynamic_update_slice(out, blk, (0, g * gsz, g * gsz))
    return out


def kernel(x, W0, g0, b0, Wx, gx, bx, W2, g2, b2, Wd2, gd2, bd2, Wd1a, gd1a,
           bd1a, Wd1b, gd1b, bd1b, Wd0, bd0, dict_val, dict_cnt):
    B, Cx, T = x.shape
    Tq = T // 4

    w0t = jnp.transpose(W0, (2, 1, 0))
    wxd = _block_diag_taps(Wx)
    w2t = jnp.transpose(W2, (2, 1, 0))
    wd2t = jnp.transpose(Wd2, (2, 0, 1))
    wd1at = jnp.transpose(Wd1a, (2, 0, 1))
    wd1bt = jnp.transpose(Wd1b, (2, 0, 1))
    wd0t = jnp.transpose(Wd0, (2, 0, 1))
    ch = jax.lax.broadcasted_iota(jnp.int32, (256, NGROUPS), 0)
    gr = jax.lax.broadcasted_iota(jnp.int32, (256, NGROUPS), 1)
    egrp = (ch // (256 // NGROUPS) == gr).astype(jnp.float32)

    h2 = pl.pallas_call(
        _encoder_body,
        grid=(B // SPP,),
        in_specs=[
            pl.BlockSpec((SPP, Cx, T), lambda b: (b, 0, 0)),
            _full((5, Cx, 256)), _full((256,)), _full((256,)),
            _full((3, 256, 256)), _full((256,)), _full((256,)),
            _full((3, 256, VQD)), _full((256, NGROUPS)),
        ],
        out_specs=pl.BlockSpec((SPP, Tq, VQD), lambda b: (b, 0, 0)),
        out_shape=jax.ShapeDtypeStruct((B, Tq, VQD), jnp.float32),
    )(x, w0t, g0, b0, wxd, gx, bx, w2t, egrp)

    zaug, idxm = pl.pallas_call(
        _argmin_body,
        in_specs=[
            _full((B, Tq, VQD)), _full((VQD,)), _full((VQD,)),
            _full((VQN, VQD)),
        ],
        out_specs=[_full((NTOK, AW)), _full((NTOK, 1))],
        out_shape=[
            jax.ShapeDtypeStruct((NTOK, AW), jnp.float32),
            jax.ShapeDtypeStruct((NTOK, 1), jnp.int32),
        ],
    )(h2, g2, b2, dict_val)

    idxf = idxm.reshape(NTOK)
    parts = _sc_scatter_add(idxf, zaug, jnp.zeros((VQN, AW), jnp.float32))

    wpad = pl.pallas_call(
        _ema_body,
        in_specs=[
            _full((SC_NC, VQN, AW)), _full((VQN, VQD)), _full((VQN, 1)),
        ],
        out_specs=_full((VQN, AW)),
        out_shape=jax.ShapeDtypeStruct((VQN, AW), jnp.float32),
    )(parts, dict_val, dict_cnt.reshape(VQN, 1))

    wj = _sc_gather(wpad, idxf)

    recon, commit = pl.pallas_call(
        _decoder_body,
        grid=(B // SPP,),
        in_specs=[
            pl.BlockSpec((SPP, Tq, AW), lambda b: (b, 0, 0)),
            pl.BlockSpec((SPP, Tq, AW), lambda b: (b, 0, 0)),
            _full((3, VQD, 256)), _full((256,)), _full((256,)),
            _full((3, 256, 256)), _full((256,)), _full((256,)),
            _full((3, 256, 256)), _full((256,)), _full((256,)),
            _full((5, 256, Cx)), _full((Cx,)), _full((256, NGROUPS)),
        ],
        out_specs=[pl.BlockSpec((SPP, Cx, T), lambda b: (b, 0, 0)),
                   _full((1, 1))],
        out_shape=[jax.ShapeDtypeStruct((B, Cx, T), jnp.float32),
                   jax.ShapeDtypeStruct((1, 1), jnp.float32)],
    )(zaug.reshape(B, Tq, AW), wj.reshape(B, Tq, AW), wd2t, gd2, bd2,
      wd1at, gd1a, bd1a, wd1bt, gd1b, bd1b, wd0t, bd0, egrp)

    return recon, commit[0, 0]
